# baseline (speedup 1.0000x reference)
import functools

import jax
import jax.numpy as jnp
from jax import lax
from jax.experimental import pallas as pl
from jax.experimental.pallas import tpu as pltpu
from jax.experimental.pallas import tpu_sc as plsc

N = 10000
D = 128
N_PAD = 10240
BM = 2560
E_CHUNK = 128
IB = 8
N_SC = 2
N_SUB = 16
N_TILES = N_SC * N_SUB
EDGE_ALIGN = N_TILES * E_CHUNK * 2 * IB
RPT = N_PAD // N_SUB


def _deg_body(ept, elast, e_hbm, deg_out, dstbuf, hist, slbuf, acc, stage):
    c = lax.axis_index("c")
    s = lax.axis_index("s")
    w = c * N_SUB + s
    zero16 = jnp.zeros((16,), jnp.float32)
    ones16 = jnp.ones((16,), jnp.float32)

    def zh(i, carry):
        hist[pl.ds(i * 16, 16)] = zero16
        return carry

    lax.fori_loop(0, N_PAD // 16, zh, 0)

    @pl.when(w < N_TILES - 1)
    def _ldfull():
        pltpu.sync_copy(e_hbm.at[1, pl.ds(w * ept, ept)], dstbuf)

    @pl.when(w == N_TILES - 1)
    def _ldlast():
        if elast > 0:
            pltpu.sync_copy(e_hbm.at[1, pl.ds((N_TILES - 1) * ept, elast)],
                            dstbuf.at[pl.ds(0, elast)])

    def body(i, carry):
        idx = dstbuf[pl.ds(i * 16, 16)]
        plsc.addupdate_scatter(hist, [idx], ones16)
        return carry

    nit = jnp.where(w == N_TILES - 1, elast // 16, ept // 16)
    lax.fori_loop(0, nit, body, 0)
    pltpu.sync_copy(hist, stage.at[s])
    plsc.subcore_barrier()

    def za(i, carry):
        acc[pl.ds(i * 16, 16)] = zero16
        return carry

    lax.fori_loop(0, RPT // 16, za, 0)
    for t in range(N_SUB):
        pltpu.sync_copy(stage.at[t, pl.ds(s * RPT, RPT)], slbuf)

        def ab(i, carry):
            acc[pl.ds(i * 16, 16)] = acc[pl.ds(i * 16, 16)] + slbuf[pl.ds(i * 16, 16)]
            return carry

        lax.fori_loop(0, RPT // 16, ab, 0)
    pltpu.sync_copy(acc, deg_out.at[c, pl.ds(s * RPT, RPT)])


@functools.lru_cache(maxsize=None)
def _deg_call(ept, elast):
    return pl.kernel(
        functools.partial(_deg_body, ept, elast),
        out_type=jax.ShapeDtypeStruct((N_SC, N_PAD), jnp.float32),
        mesh=plsc.VectorSubcoreMesh(core_axis_name="c", subcore_axis_name="s"),
        compiler_params=pltpu.CompilerParams(needs_layout_passes=False),
        scratch_types=[
            pltpu.VMEM((ept,), jnp.int32),
            pltpu.VMEM((N_PAD,), jnp.float32),
            pltpu.VMEM((RPT,), jnp.float32),
            pltpu.VMEM((RPT,), jnp.float32),
            pltpu.VMEM_SHARED((N_SUB, N_PAD), jnp.float32),
        ],
    )


def _mm_body(x_ref, w_ref, deg_ref, g_ref):
    i = pl.program_id(0)
    h = jnp.dot(x_ref[...], w_ref[...], preferred_element_type=jnp.float32)
    deg = deg_ref[0, pl.ds(i * BM, BM)] + deg_ref[1, pl.ds(i * BM, BM)] + 1.0
    dis = lax.rsqrt(deg)
    g_ref[...] = h * dis[:, None]


@functools.lru_cache(maxsize=None)
def _mm_call():
    return pl.pallas_call(
        _mm_body,
        grid=(N_PAD // BM,),
        in_specs=[
            pl.BlockSpec((BM, D), lambda i: (i, 0)),
            pl.BlockSpec((D, D), lambda i: (0, 0)),
            pl.BlockSpec((N_SC, N_PAD), lambda i: (0, 0)),
        ],
        out_specs=pl.BlockSpec((BM, D), lambda i: (i, 0)),
        out_shape=jax.ShapeDtypeStruct((N_PAD, D), jnp.float32),
    )


def _scatter_body(nb, rc, e_hbm, mix_hbm, psrc_hbm, pdst_hbm, g_hbm, part_hbm,
                  si0, di0, si1, di1, rows0, rows1, agg,
                  isem0, isem1, gsem0, gsem1, ssem0, ssem1):
    c = lax.axis_index("c")
    s = lax.axis_index("s")
    w = c * N_SUB + s
    rbase = s * RPT
    zero16 = jnp.zeros((16,), jnp.float32)

    @pl.when(c == 0)
    def _initg():
        pltpu.sync_copy(g_hbm.at[pl.ds(rbase, RPT)], agg.at[pl.ds(rbase, RPT)])

    @pl.when(c == 1)
    def _initz():
        def zr(r, carry):
            for jj in range(D // 16):
                rows0[r, pl.ds(jj * 16, 16)] = zero16
            return carry

        lax.fori_loop(0, E_CHUNK, zr, 0)
        for t in range(RPT // E_CHUNK):
            pltpu.sync_copy(rows0, agg.at[pl.ds(rbase + t * E_CHUNK, E_CHUNK)])

    plsc.subcore_barrier()
    nkc = nb * IB
    chunk0 = w * nkc
    rb = (rc // IB) * IB

    ibufs = ((si0, di0, isem0), (si1, di1, isem1))
    rslots = ((rows0, gsem0, ssem0), (rows1, gsem1, ssem1))

    def fire_idx(b, ib):
        si, di, isem = ib
        cr = chunk0 + b * IB

        @pl.when(cr < rb)
        def _real():
            pltpu.async_copy(e_hbm.at[0, pl.ds(cr, IB)], si, isem)
            pltpu.async_copy(e_hbm.at[1, pl.ds(cr, IB)], di, isem)

        @pl.when(cr == rb)
        def _mixed():
            pltpu.async_copy(mix_hbm.at[0], si, isem)
            pltpu.async_copy(mix_hbm.at[1], di, isem)

        @pl.when(cr > rb)
        def _pad():
            pltpu.async_copy(psrc_hbm.at[pl.ds(cr - rb - IB, IB)], si, isem)
            pltpu.async_copy(pdst_hbm.at[pl.ds(cr - rb - IB, IB)], di, isem)

    def wait_idx(b, ib):
        si, di, isem = ib
        pltpu.make_async_copy(e_hbm.at[0, pl.ds(0, IB)], si, isem).wait()
        pltpu.make_async_copy(e_hbm.at[1, pl.ds(0, IB)], di, isem).wait()

    def fire_gather(ib, row, rs):
        pltpu.async_copy(g_hbm.at[ib[0].at[row]], rs[0], rs[1])

    def wait_gather(ib, row, rs):
        pltpu.make_async_copy(g_hbm.at[ib[0].at[row]], rs[0], rs[1]).wait()

    def fire_scatter(ib, row, rs):
        pltpu.async_copy(rs[0], agg.at[ib[1].at[row]], rs[2], add=True)

    def wait_scatter(ib, row, rs):
        pltpu.make_async_copy(rs[0], agg.at[ib[1].at[row]], rs[2]).wait()

    fire_idx(jnp.int32(0), ibufs[0])
    wait_idx(jnp.int32(0), ibufs[0])
    fire_gather(ibufs[0], 0, rslots[0])

    def body(m, carry):
        b0 = 2 * m
        k0 = b0 * IB
        for j in range(2 * IB):
            cur_ib = ibufs[j // IB]
            cur_row = j % IB
            cur_rs = rslots[j % 2]
            wait_gather(cur_ib, cur_row, cur_rs)
            fire_scatter(cur_ib, cur_row, cur_rs)
            if j == 0:
                pib = ibufs[1]

                @pl.when(k0 > 0)
                def _drain():
                    wait_scatter(pib, IB - 1, rslots[1])
            else:
                pj = j - 1
                wait_scatter(ibufs[pj // IB], pj % IB, rslots[pj % 2])
            if j == 0:
                fire_idx(b0 + 1, ibufs[1])
            elif j == IB:
                @pl.when(b0 + 2 < nb)
                def _refresh():
                    fire_idx(b0 + 2, ibufs[0])
            nj = j + 1
            if nj < 2 * IB:
                nib = ibufs[nj // IB]
                if nj == IB:
                    wait_idx(b0 + 1, ibufs[1])
                fire_gather(nib, nj % IB, rslots[nj % 2])
            else:
                @pl.when(b0 + 2 < nb)
                def _nextblock():
                    wait_idx(b0 + 2, ibufs[0])
                    fire_gather(ibufs[0], 0, rslots[0])

        return carry

    lax.fori_loop(0, nb // 2, body, 0)
    wait_scatter(ibufs[1], IB - 1, rslots[1])
    plsc.subcore_barrier()
    pltpu.sync_copy(agg.at[pl.ds(rbase, RPT)], part_hbm.at[c, pl.ds(rbase, RPT)])


@functools.lru_cache(maxsize=None)
def _scatter_call(nb, rc):
    assert nb % 2 == 0 and IB % 2 == 0
    return pl.kernel(
        functools.partial(_scatter_body, nb, rc),
        out_type=jax.ShapeDtypeStruct((N_SC, N_PAD, D), jnp.float32),
        mesh=plsc.VectorSubcoreMesh(core_axis_name="c", subcore_axis_name="s"),
        compiler_params=pltpu.CompilerParams(needs_layout_passes=False),
        scratch_types=[
            pltpu.VMEM((IB, E_CHUNK), jnp.int32),
            pltpu.VMEM((IB, E_CHUNK), jnp.int32),
            pltpu.VMEM((IB, E_CHUNK), jnp.int32),
            pltpu.VMEM((IB, E_CHUNK), jnp.int32),
            pltpu.VMEM((E_CHUNK, D), jnp.float32),
            pltpu.VMEM((E_CHUNK, D), jnp.float32),
            pltpu.VMEM_SHARED((N_PAD, D), jnp.float32),
            pltpu.SemaphoreType.DMA,
            pltpu.SemaphoreType.DMA,
            pltpu.SemaphoreType.DMA,
            pltpu.SemaphoreType.DMA,
            pltpu.SemaphoreType.DMA,
            pltpu.SemaphoreType.DMA,
        ],
    )


def _final_body(p_ref, deg_ref, b_ref, pw_ref, o_ref):
    i = pl.program_id(0)
    deg = deg_ref[0, pl.ds(i * BM, BM)] + deg_ref[1, pl.ds(i * BM, BM)] + 1.0
    dis = lax.rsqrt(deg)
    v = (p_ref[0] + p_ref[1]) * dis[:, None] + b_ref[0][None, :]
    o_ref[...] = jnp.where(v >= 0, v, pw_ref[0][None, :] * v)


@functools.lru_cache(maxsize=None)
def _final_call():
    return pl.pallas_call(
        _final_body,
        grid=(pl.cdiv(N, BM),),
        in_specs=[
            pl.BlockSpec((N_SC, BM, D), lambda i: (0, i, 0)),
            pl.BlockSpec((N_SC, N_PAD), lambda i: (0, 0)),
            pl.BlockSpec((1, D), lambda i: (0, 0)),
            pl.BlockSpec((1, D), lambda i: (0, 0)),
        ],
        out_specs=pl.BlockSpec((BM, D), lambda i: (i, 0)),
        out_shape=jax.ShapeDtypeStruct((N, D), jnp.float32),
    )


def kernel(x, edge_index, W, b, prelu_weight):
    e = edge_index.shape[1]
    assert e % E_CHUNK == 0
    rc = e // E_CHUNK
    rem = rc % IB
    e_pad = ((e + EDGE_ALIGN - 1) // EDGE_ALIGN) * EDGE_ALIGN
    padc = e_pad // E_CHUNK - rc
    assert padc >= IB - rem
    ept = e_pad // N_TILES
    elast = e - (N_TILES - 1) * ept
    assert 0 <= elast and elast % 16 == 0
    pad_i = jnp.arange(padc * E_CHUNK, dtype=jnp.int32)
    psrc = (pad_i % N_PAD).reshape(padc, E_CHUNK)
    pdst = (N + pad_i % (N_PAD - N)).reshape(padc, E_CHUNK)
    e1 = edge_index.astype(jnp.int32)
    e2 = e1.reshape(2, rc, E_CHUNK)
    mix = jnp.concatenate(
        [e2[:, rc - rem:],
         jnp.stack([psrc[:IB - rem], pdst[:IB - rem]])], axis=1)

    deg2 = _deg_call(ept, elast)(e1)
    g = _mm_call()(x, W.astype(jnp.float32), deg2)
    nb = ept // (E_CHUNK * IB)
    parts = _scatter_call(nb, rc)(e2, mix, psrc[IB - rem:], pdst[IB - rem:], g)
    return _final_call()(parts, deg2,
                         b.reshape(1, D).astype(jnp.float32),
                         prelu_weight.reshape(1, D).astype(jnp.float32))


# speedup vs baseline: 1.0107x; 1.0107x over previous
import functools

import jax
import jax.numpy as jnp
from jax import lax
from jax.experimental import pallas as pl
from jax.experimental.pallas import tpu as pltpu
from jax.experimental.pallas import tpu_sc as plsc

N = 10000
D = 128
N_PAD = 10240
BM = 5120
E_CHUNK = 128
IB = 8
N_SC = 2
N_SUB = 16
N_TILES = N_SC * N_SUB
EDGE_ALIGN = N_TILES * E_CHUNK * 2 * IB
RPT = N_PAD // N_SUB


def _deg_body(ept, elast, e_hbm, deg_out, dstbuf, hist, slbuf, acc, stage):
    c = lax.axis_index("c")
    s = lax.axis_index("s")
    w = c * N_SUB + s
    zero16 = jnp.zeros((16,), jnp.float32)
    ones16 = jnp.ones((16,), jnp.float32)

    def zh(i, carry):
        hist[pl.ds(i * 16, 16)] = zero16
        return carry

    lax.fori_loop(0, N_PAD // 16, zh, 0)

    @pl.when(w < N_TILES - 1)
    def _ldfull():
        pltpu.sync_copy(e_hbm.at[1, pl.ds(w * ept, ept)], dstbuf)

    @pl.when(w == N_TILES - 1)
    def _ldlast():
        if elast > 0:
            pltpu.sync_copy(e_hbm.at[1, pl.ds((N_TILES - 1) * ept, elast)],
                            dstbuf.at[pl.ds(0, elast)])

    def body(i, carry):
        idx = dstbuf[pl.ds(i * 16, 16)]
        plsc.addupdate_scatter(hist, [idx], ones16)
        return carry

    nit = jnp.where(w == N_TILES - 1, elast // 16, ept // 16)
    lax.fori_loop(0, nit, body, 0)
    pltpu.sync_copy(hist, stage.at[s])
    plsc.subcore_barrier()

    def za(i, carry):
        acc[pl.ds(i * 16, 16)] = zero16
        return carry

    lax.fori_loop(0, RPT // 16, za, 0)
    for t in range(N_SUB):
        pltpu.sync_copy(stage.at[t, pl.ds(s * RPT, RPT)], slbuf)

        def ab(i, carry):
            acc[pl.ds(i * 16, 16)] = acc[pl.ds(i * 16, 16)] + slbuf[pl.ds(i * 16, 16)]
            return carry

        lax.fori_loop(0, RPT // 16, ab, 0)
    pltpu.sync_copy(acc, deg_out.at[c, pl.ds(s * RPT, RPT)])


@functools.lru_cache(maxsize=None)
def _deg_call(ept, elast):
    return pl.kernel(
        functools.partial(_deg_body, ept, elast),
        out_type=jax.ShapeDtypeStruct((N_SC, N_PAD), jnp.float32),
        mesh=plsc.VectorSubcoreMesh(core_axis_name="c", subcore_axis_name="s"),
        compiler_params=pltpu.CompilerParams(needs_layout_passes=False),
        scratch_types=[
            pltpu.VMEM((ept,), jnp.int32),
            pltpu.VMEM((N_PAD,), jnp.float32),
            pltpu.VMEM((RPT,), jnp.float32),
            pltpu.VMEM((RPT,), jnp.float32),
            pltpu.VMEM_SHARED((N_SUB, N_PAD), jnp.float32),
        ],
    )


def _mm_body(x_ref, w_ref, deg_ref, g_ref):
    i = pl.program_id(0)
    h = jnp.dot(x_ref[...], w_ref[...], preferred_element_type=jnp.float32)
    deg = deg_ref[0, pl.ds(i * BM, BM)] + deg_ref[1, pl.ds(i * BM, BM)] + 1.0
    dis = lax.rsqrt(deg)
    g_ref[...] = h * dis[:, None]


@functools.lru_cache(maxsize=None)
def _mm_call():
    return pl.pallas_call(
        _mm_body,
        grid=(N_PAD // BM,),
        in_specs=[
            pl.BlockSpec((BM, D), lambda i: (i, 0)),
            pl.BlockSpec((D, D), lambda i: (0, 0)),
            pl.BlockSpec((N_SC, N_PAD), lambda i: (0, 0)),
        ],
        out_specs=pl.BlockSpec((BM, D), lambda i: (i, 0)),
        out_shape=jax.ShapeDtypeStruct((N_PAD, D), jnp.float32),
    )


def _scatter_body(nb, rc, e_hbm, mix_hbm, psrc_hbm, pdst_hbm, g_hbm, part_hbm,
                  si0, di0, si1, di1, rows0, rows1, agg,
                  isem0, isem1, gsem0, gsem1, ssem0, ssem1):
    c = lax.axis_index("c")
    s = lax.axis_index("s")
    w = c * N_SUB + s
    rbase = s * RPT
    zero16 = jnp.zeros((16,), jnp.float32)

    @pl.when(c == 0)
    def _initg():
        pltpu.sync_copy(g_hbm.at[pl.ds(rbase, RPT)], agg.at[pl.ds(rbase, RPT)])

    @pl.when(c == 1)
    def _initz():
        def zr(r, carry):
            for jj in range(D // 16):
                rows0[r, pl.ds(jj * 16, 16)] = zero16
            return carry

        lax.fori_loop(0, E_CHUNK, zr, 0)
        for t in range(RPT // E_CHUNK):
            pltpu.sync_copy(rows0, agg.at[pl.ds(rbase + t * E_CHUNK, E_CHUNK)])

    plsc.subcore_barrier()
    nkc = nb * IB
    chunk0 = w * nkc
    rb = (rc // IB) * IB

    ibufs = ((si0, di0, isem0), (si1, di1, isem1))
    rslots = ((rows0, gsem0, ssem0), (rows1, gsem1, ssem1))

    def fire_idx(b, ib):
        si, di, isem = ib
        cr = chunk0 + b * IB

        @pl.when(cr < rb)
        def _real():
            pltpu.async_copy(e_hbm.at[0, pl.ds(cr, IB)], si, isem)
            pltpu.async_copy(e_hbm.at[1, pl.ds(cr, IB)], di, isem)

        @pl.when(cr == rb)
        def _mixed():
            pltpu.async_copy(mix_hbm.at[0], si, isem)
            pltpu.async_copy(mix_hbm.at[1], di, isem)

        @pl.when(cr > rb)
        def _pad():
            pltpu.async_copy(psrc_hbm.at[pl.ds(cr - rb - IB, IB)], si, isem)
            pltpu.async_copy(pdst_hbm.at[pl.ds(cr - rb - IB, IB)], di, isem)

    def wait_idx(b, ib):
        si, di, isem = ib
        pltpu.make_async_copy(e_hbm.at[0, pl.ds(0, IB)], si, isem).wait()
        pltpu.make_async_copy(e_hbm.at[1, pl.ds(0, IB)], di, isem).wait()

    def fire_gather(ib, row, rs):
        pltpu.async_copy(g_hbm.at[ib[0].at[row]], rs[0], rs[1])

    def wait_gather(ib, row, rs):
        pltpu.make_async_copy(g_hbm.at[ib[0].at[row]], rs[0], rs[1]).wait()

    def fire_scatter(ib, row, rs):
        pltpu.async_copy(rs[0], agg.at[ib[1].at[row]], rs[2], add=True)

    def wait_scatter(ib, row, rs):
        pltpu.make_async_copy(rs[0], agg.at[ib[1].at[row]], rs[2]).wait()

    fire_idx(jnp.int32(0), ibufs[0])
    wait_idx(jnp.int32(0), ibufs[0])
    fire_gather(ibufs[0], 0, rslots[0])

    def body(m, carry):
        b0 = 2 * m
        k0 = b0 * IB
        for j in range(2 * IB):
            cur_ib = ibufs[j // IB]
            cur_row = j % IB
            cur_rs = rslots[j % 2]
            wait_gather(cur_ib, cur_row, cur_rs)
            fire_scatter(cur_ib, cur_row, cur_rs)
            if j == 0:
                pib = ibufs[1]

                @pl.when(k0 > 0)
                def _drain():
                    wait_scatter(pib, IB - 1, rslots[1])
            else:
                pj = j - 1
                wait_scatter(ibufs[pj // IB], pj % IB, rslots[pj % 2])
            if j == 0:
                fire_idx(b0 + 1, ibufs[1])
            elif j == IB:
                @pl.when(b0 + 2 < nb)
                def _refresh():
                    fire_idx(b0 + 2, ibufs[0])
            nj = j + 1
            if nj < 2 * IB:
                nib = ibufs[nj // IB]
                if nj == IB:
                    wait_idx(b0 + 1, ibufs[1])
                fire_gather(nib, nj % IB, rslots[nj % 2])
            else:
                @pl.when(b0 + 2 < nb)
                def _nextblock():
                    wait_idx(b0 + 2, ibufs[0])
                    fire_gather(ibufs[0], 0, rslots[0])

        return carry

    lax.fori_loop(0, nb // 2, body, 0)
    wait_scatter(ibufs[1], IB - 1, rslots[1])
    plsc.subcore_barrier()
    pltpu.sync_copy(agg.at[pl.ds(rbase, RPT)], part_hbm.at[c, pl.ds(rbase, RPT)])


@functools.lru_cache(maxsize=None)
def _scatter_call(nb, rc):
    assert nb % 2 == 0 and IB % 2 == 0
    return pl.kernel(
        functools.partial(_scatter_body, nb, rc),
        out_type=jax.ShapeDtypeStruct((N_SC, N_PAD, D), jnp.float32),
        mesh=plsc.VectorSubcoreMesh(core_axis_name="c", subcore_axis_name="s"),
        compiler_params=pltpu.CompilerParams(needs_layout_passes=False),
        scratch_types=[
            pltpu.VMEM((IB, E_CHUNK), jnp.int32),
            pltpu.VMEM((IB, E_CHUNK), jnp.int32),
            pltpu.VMEM((IB, E_CHUNK), jnp.int32),
            pltpu.VMEM((IB, E_CHUNK), jnp.int32),
            pltpu.VMEM((E_CHUNK, D), jnp.float32),
            pltpu.VMEM((E_CHUNK, D), jnp.float32),
            pltpu.VMEM_SHARED((N_PAD, D), jnp.float32),
            pltpu.SemaphoreType.DMA,
            pltpu.SemaphoreType.DMA,
            pltpu.SemaphoreType.DMA,
            pltpu.SemaphoreType.DMA,
            pltpu.SemaphoreType.DMA,
            pltpu.SemaphoreType.DMA,
        ],
    )


def _final_body(p_ref, deg_ref, b_ref, pw_ref, o_ref):
    i = pl.program_id(0)
    deg = deg_ref[0, pl.ds(i * BM, BM)] + deg_ref[1, pl.ds(i * BM, BM)] + 1.0
    dis = lax.rsqrt(deg)
    v = (p_ref[0] + p_ref[1]) * dis[:, None] + b_ref[0][None, :]
    o_ref[...] = jnp.where(v >= 0, v, pw_ref[0][None, :] * v)


@functools.lru_cache(maxsize=None)
def _final_call():
    return pl.pallas_call(
        _final_body,
        grid=(pl.cdiv(N, BM),),
        in_specs=[
            pl.BlockSpec((N_SC, BM, D), lambda i: (0, i, 0)),
            pl.BlockSpec((N_SC, N_PAD), lambda i: (0, 0)),
            pl.BlockSpec((1, D), lambda i: (0, 0)),
            pl.BlockSpec((1, D), lambda i: (0, 0)),
        ],
        out_specs=pl.BlockSpec((BM, D), lambda i: (i, 0)),
        out_shape=jax.ShapeDtypeStruct((N, D), jnp.float32),
    )


def kernel(x, edge_index, W, b, prelu_weight):
    e = edge_index.shape[1]
    assert e % E_CHUNK == 0
    rc = e // E_CHUNK
    rem = rc % IB
    e_pad = ((e + EDGE_ALIGN - 1) // EDGE_ALIGN) * EDGE_ALIGN
    padc = e_pad // E_CHUNK - rc
    assert padc >= IB - rem
    ept = e_pad // N_TILES
    elast = e - (N_TILES - 1) * ept
    assert 0 <= elast and elast % 16 == 0
    pad_i = jnp.arange(padc * E_CHUNK, dtype=jnp.int32)
    psrc = (pad_i % N_PAD).reshape(padc, E_CHUNK)
    pdst = (N + pad_i % (N_PAD - N)).reshape(padc, E_CHUNK)
    e1 = edge_index.astype(jnp.int32)
    e2 = e1.reshape(2, rc, E_CHUNK)
    mix = jnp.concatenate(
        [e2[:, rc - rem:],
         jnp.stack([psrc[:IB - rem], pdst[:IB - rem]])], axis=1)

    deg2 = _deg_call(ept, elast)(e1)
    g = _mm_call()(x, W.astype(jnp.float32), deg2)
    nb = ept // (E_CHUNK * IB)
    parts = _scatter_call(nb, rc)(e2, mix, psrc[IB - rem:], pdst[IB - rem:], g)
    return _final_call()(parts, deg2,
                         b.reshape(1, D).astype(jnp.float32),
                         prelu_weight.reshape(1, D).astype(jnp.float32))
